# Initial kernel scaffold; baseline (speedup 1.0000x reference)
import functools

import jax
import jax.numpy as jnp
from jax import lax
from jax.experimental import pallas as pl
from jax.experimental.pallas import tpu as pltpu
from jax.experimental.pallas import tpu_sc as plsc

B, C_IN, C_OUT, E, K = 1, 64, 64, 320000, 5
EPS = 1e-05

NC, NS = 2, 16
NW = NC * NS
ROWS_PER_W = K * E // NW
CHUNK = 128
NB = 6
NFULL = ROWS_PER_W // CHUNK
NGROUPS = NFULL // NB
TAIL = ROWS_PER_W - NFULL * CHUNK


def _sc_gather(feT, idx_flat):
    mesh = plsc.VectorSubcoreMesh(core_axis_name="c", subcore_axis_name="s")

    @functools.partial(
        pl.kernel,
        out_type=jax.ShapeDtypeStruct((K * E, C_IN), feT.dtype),
        mesh=mesh,
        scratch_types=[
            pltpu.VMEM((NB, CHUNK), jnp.int32),
            pltpu.VMEM((NB, CHUNK, C_IN), feT.dtype),
            pltpu.SemaphoreType.DMA,
            pltpu.SemaphoreType.DMA,
        ],
    )
    def k(feT_hbm, idx_hbm, out_hbm, idx_v, rows_v, gsem, wsem):
        wid = lax.axis_index("s") * NC + lax.axis_index("c")
        base = wid * ROWS_PER_W

        def group(g, carry):
            starts = [base + (g * NB + b) * CHUNK for b in range(NB)]
            gh = []
            for b in range(NB):
                pltpu.sync_copy(idx_hbm.at[pl.ds(starts[b], CHUNK)], idx_v.at[b])
                gh.append(pltpu.async_copy(feT_hbm.at[idx_v.at[b]], rows_v.at[b], gsem))
            wh = []
            for b in range(NB):
                gh[b].wait()
                wh.append(pltpu.async_copy(rows_v.at[b], out_hbm.at[pl.ds(starts[b], CHUNK)], wsem))
            for b in range(NB):
                wh[b].wait()
            return carry

        lax.fori_loop(0, NGROUPS, group, 0, unroll=False)

        tstart = base + NFULL * CHUNK
        pltpu.sync_copy(idx_hbm.at[pl.ds(tstart, TAIL)], idx_v.at[0, pl.ds(0, TAIL)])
        pltpu.async_copy(feT_hbm.at[idx_v.at[0, pl.ds(0, TAIL)]],
                         rows_v.at[0, pl.ds(0, TAIL)], gsem).wait()
        pltpu.async_copy(rows_v.at[0, pl.ds(0, TAIL)],
                         out_hbm.at[pl.ds(tstart, TAIL)], wsem).wait()

    return k(feT, idx_flat)


ET = 1280


def _tc_transpose(fe2):
    def body(fe_ref, o_ref):
        o_ref[...] = fe_ref[...].T

    return pl.pallas_call(
        body,
        grid=(E // ET,),
        in_specs=[pl.BlockSpec((C_IN, ET), lambda i: (0, i))],
        out_specs=pl.BlockSpec((ET, C_IN), lambda i: (i, 0)),
        out_shape=jax.ShapeDtypeStruct((E, C_IN), jnp.float32),
    )(fe2)


def _tc_conv(fg, wmat):
    def body(fg_ref, w_ref, raw_ref, stats_ref):
        i = pl.program_id(0)
        f0 = fg_ref[0]
        f1 = fg_ref[1]
        f2 = fg_ref[2]
        f3 = fg_ref[3]
        f4 = fg_ref[4]
        fn = jnp.concatenate(
            [f0, f1 + f3, f2 + f4, jnp.abs(f1 - f3), jnp.abs(f2 - f4)], axis=1)
        out = lax.dot_general(w_ref[...], fn, (((1,), (1,)), ((), ())),
                              preferred_element_type=jnp.float32)
        raw_ref[...] = out

        @pl.when(i == 0)
        def _init():
            stats_ref[...] = jnp.zeros_like(stats_ref)

        s = jnp.sum(out, axis=1, keepdims=True)
        sq = jnp.sum(out * out, axis=1, keepdims=True)
        stats_ref[...] += jnp.concatenate([s, sq], axis=1)

    return pl.pallas_call(
        body,
        grid=(E // ET,),
        in_specs=[pl.BlockSpec((K, ET, C_IN), lambda i: (0, i, 0)),
                  pl.BlockSpec((C_OUT, K * C_IN), lambda i: (0, 0))],
        out_specs=[pl.BlockSpec((C_OUT, ET), lambda i: (0, i)),
                   pl.BlockSpec((C_OUT, 2), lambda i: (0, 0))],
        out_shape=[jax.ShapeDtypeStruct((C_OUT, E), jnp.float32),
                   jax.ShapeDtypeStruct((C_OUT, 2), jnp.float32)],
    )(fg, wmat)


def _tc_norm(raw, stats):
    inv_e = 1.0 / E

    def body(raw_ref, st_ref, o_ref):
        st = st_ref[...]
        mean = st[:, 0:1] * inv_e
        ex2 = st[:, 1:2] * inv_e
        var = jnp.maximum(ex2 - mean * mean, 0.0)
        rstd = 1.0 / (jnp.sqrt(var) + EPS)
        o_ref[...] = jnp.maximum((raw_ref[...] - mean) * rstd, 0.0)

    return pl.pallas_call(
        body,
        grid=(E // ET,),
        in_specs=[pl.BlockSpec((C_OUT, ET), lambda i: (0, i)),
                  pl.BlockSpec((C_OUT, 2), lambda i: (0, 0))],
        out_specs=pl.BlockSpec((C_OUT, ET), lambda i: (0, i)),
        out_shape=jax.ShapeDtypeStruct((C_OUT, E), jnp.float32),
    )(raw, stats)


def kernel(fe, gemm_edges, W, b):
    del b
    fe2 = fe[0]
    idx_flat = gemm_edges[0].T.reshape(K * E)
    wmat = jnp.transpose(W, (0, 2, 1)).reshape(C_OUT, K * C_IN)
    feT = _tc_transpose(fe2)
    fg = _sc_gather(feT, idx_flat).reshape(K, E, C_IN)
    raw, stats = _tc_conv(fg, wmat)
    out = _tc_norm(raw, stats)
    return out[None]


# speedup vs baseline: 2.8997x; 2.8997x over previous
import functools

import jax
import jax.numpy as jnp
from jax import lax
from jax.experimental import pallas as pl
from jax.experimental.pallas import tpu as pltpu
from jax.experimental.pallas import tpu_sc as plsc

B, C_IN, C_OUT, E, K = 1, 64, 64, 320000, 5
EPS = 1e-05

NC, NS = 2, 16
NW = NC * NS
ROWS_PER_W = K * E // NW
CHUNK = 128
NB = 6
NFULL = ROWS_PER_W // CHUNK
NGROUPS = NFULL // NB
TAIL = ROWS_PER_W - NFULL * CHUNK


def _sc_gather(feT, idx_flat):
    mesh = plsc.VectorSubcoreMesh(core_axis_name="c", subcore_axis_name="s")

    @functools.partial(
        pl.kernel,
        out_type=jax.ShapeDtypeStruct((K * E, C_IN), feT.dtype),
        mesh=mesh,
        scratch_types=[
            pltpu.VMEM((NB, CHUNK), jnp.int32),
            pltpu.VMEM((NB, CHUNK, C_IN), feT.dtype),
            pltpu.SemaphoreType.DMA,
            pltpu.SemaphoreType.DMA,
        ],
        compiler_params=pltpu.CompilerParams(use_tc_tiling_on_sc=False),
    )
    def k(feT_hbm, idx_hbm, out_hbm, idx_v, rows_v, gsem, wsem):
        wid = lax.axis_index("s") * NC + lax.axis_index("c")
        base = wid * ROWS_PER_W

        def group(g, carry):
            starts = [base + (g * NB + b) * CHUNK for b in range(NB)]
            gh = []
            for b in range(NB):
                pltpu.sync_copy(idx_hbm.at[pl.ds(starts[b], CHUNK)], idx_v.at[b])
                gh.append(pltpu.async_copy(feT_hbm.at[idx_v.at[b]], rows_v.at[b], gsem))
            wh = []
            for b in range(NB):
                gh[b].wait()
                wh.append(pltpu.async_copy(rows_v.at[b], out_hbm.at[pl.ds(starts[b], CHUNK)], wsem))
            for b in range(NB):
                wh[b].wait()
            return carry

        lax.fori_loop(0, NGROUPS, group, 0, unroll=False)

        tstart = base + NFULL * CHUNK
        pltpu.sync_copy(idx_hbm.at[pl.ds(tstart, TAIL)], idx_v.at[0, pl.ds(0, TAIL)])
        pltpu.async_copy(feT_hbm.at[idx_v.at[0, pl.ds(0, TAIL)]],
                         rows_v.at[0, pl.ds(0, TAIL)], gsem).wait()
        pltpu.async_copy(rows_v.at[0, pl.ds(0, TAIL)],
                         out_hbm.at[pl.ds(tstart, TAIL)], wsem).wait()

    return k(feT, idx_flat)


ET = 1280


def _tc_transpose(fe2):
    def body(fe_ref, o_ref):
        o_ref[...] = fe_ref[...].T

    return pl.pallas_call(
        body,
        grid=(E // ET,),
        in_specs=[pl.BlockSpec((C_IN, ET), lambda i: (0, i))],
        out_specs=pl.BlockSpec((ET, C_IN), lambda i: (i, 0)),
        out_shape=jax.ShapeDtypeStruct((E, C_IN), jnp.float32),
    )(fe2)


def _tc_conv(fg, wmat):
    def body(fg_ref, w_ref, raw_ref, stats_ref):
        i = pl.program_id(0)
        f0 = fg_ref[0]
        f1 = fg_ref[1]
        f2 = fg_ref[2]
        f3 = fg_ref[3]
        f4 = fg_ref[4]
        fn = jnp.concatenate(
            [f0, f1 + f3, f2 + f4, jnp.abs(f1 - f3), jnp.abs(f2 - f4)], axis=1)
        out = lax.dot_general(w_ref[...], fn, (((1,), (1,)), ((), ())),
                              preferred_element_type=jnp.float32)
        raw_ref[...] = out

        @pl.when(i == 0)
        def _init():
            stats_ref[...] = jnp.zeros_like(stats_ref)

        s = jnp.sum(out, axis=1, keepdims=True)
        sq = jnp.sum(out * out, axis=1, keepdims=True)
        stats_ref[...] += jnp.concatenate([s, sq], axis=1)

    return pl.pallas_call(
        body,
        grid=(E // ET,),
        in_specs=[pl.BlockSpec((K, ET, C_IN), lambda i: (0, i, 0)),
                  pl.BlockSpec((C_OUT, K * C_IN), lambda i: (0, 0))],
        out_specs=[pl.BlockSpec((C_OUT, ET), lambda i: (0, i)),
                   pl.BlockSpec((C_OUT, 2), lambda i: (0, 0))],
        out_shape=[jax.ShapeDtypeStruct((C_OUT, E), jnp.float32),
                   jax.ShapeDtypeStruct((C_OUT, 2), jnp.float32)],
    )(fg, wmat)


def _tc_norm(raw, stats):
    inv_e = 1.0 / E

    def body(raw_ref, st_ref, o_ref):
        st = st_ref[...]
        mean = st[:, 0:1] * inv_e
        ex2 = st[:, 1:2] * inv_e
        var = jnp.maximum(ex2 - mean * mean, 0.0)
        rstd = 1.0 / (jnp.sqrt(var) + EPS)
        o_ref[...] = jnp.maximum((raw_ref[...] - mean) * rstd, 0.0)

    return pl.pallas_call(
        body,
        grid=(E // ET,),
        in_specs=[pl.BlockSpec((C_OUT, ET), lambda i: (0, i)),
                  pl.BlockSpec((C_OUT, 2), lambda i: (0, 0))],
        out_specs=pl.BlockSpec((C_OUT, ET), lambda i: (0, i)),
        out_shape=jax.ShapeDtypeStruct((C_OUT, E), jnp.float32),
    )(raw, stats)


def kernel(fe, gemm_edges, W, b):
    del b
    fe2 = fe[0]
    idx_flat = gemm_edges[0].T.reshape(K * E)
    wmat = jnp.transpose(W, (0, 2, 1)).reshape(C_OUT, K * C_IN)
    feT = _tc_transpose(fe2)
    fg = _sc_gather(feT, idx_flat).reshape(K, E, C_IN)
    raw, stats = _tc_conv(fg, wmat)
    out = _tc_norm(raw, stats)
    return out[None]
